# Initial kernel scaffold; baseline (speedup 1.0000x reference)
import jax
import jax.numpy as jnp
from jax.experimental import pallas as pl


def kernel(x, edge_index, edge_attr, W1a, b1a, W1b, b1b, root1, bias1, g1, be1, W2a, b2a, W2b, b2b, root2, bias2, g2, be2, W3a, b3a, W3b, b3b, root3, bias3, g3, be3):
    raise NotImplementedError("write your pallas kernel here")


# speedup vs baseline: 3.8294x; 3.8294x over previous
import functools

import numpy as np
import jax
import jax.numpy as jnp
from jax import lax
from jax.experimental import pallas as pl
from jax.experimental.pallas import tpu as pltpu
from jax.experimental.pallas import tpu_sc as plsc

N = 10000
E = 160000
NC, NS = 2, 16
NW = NC * NS
K = 125
C = E // (NW * K)
PW = C * K
CPB = 8
NB = C // CPB
NPAD = 10240
RP = NPAD // NS
DOUT = 16
EPS = 1e-5
_MESH = dict(core_axis_name="c", subcore_axis_name="s")


def _gather(table, idx_w, D):
    mesh = plsc.VectorSubcoreMesh(**_MESH)

    @functools.partial(
        pl.kernel,
        out_type=jax.ShapeDtypeStruct((E, D), jnp.float32),
        mesh=mesh,
        compiler_params=pltpu.CompilerParams(use_tc_tiling_on_sc=False),
        scratch_types=[
            pltpu.VMEM((C, K), jnp.int32),
            pltpu.VMEM((CPB * K, D), jnp.float32),
            pltpu.SemaphoreType.DMA,
        ],
    )
    def gk(table_hbm, idx_hbm, out_hbm, idx_v, rows_v, sem):
        wid = lax.axis_index("s") * NC + lax.axis_index("c")
        base = wid * PW
        pltpu.sync_copy(idx_hbm.at[wid], idx_v)

        @pl.loop(0, NB)
        def _(cc):
            @pl.loop(0, CPB)
            def _(t):
                pltpu.async_copy(table_hbm.at[idx_v.at[cc * CPB + t]],
                                 rows_v.at[pl.ds(t * K, K)], sem).wait()

            pltpu.sync_copy(rows_v, out_hbm.at[pl.ds(base + cc * (CPB * K),
                                                     CPB * K)])

    return gk(table, idx_w)


def _scatter(msg, idx_w, zinit):
    mesh = plsc.VectorSubcoreMesh(**_MESH)

    @functools.partial(
        pl.kernel,
        out_type=jax.ShapeDtypeStruct((NC, NPAD, DOUT), jnp.float32),
        mesh=mesh,
        compiler_params=pltpu.CompilerParams(use_tc_tiling_on_sc=False),
        scratch_types=[
            pltpu.VMEM((C, K), jnp.int32),
            pltpu.VMEM((PW, DOUT), jnp.float32),
            pltpu.VMEM_SHARED((NPAD, DOUT), jnp.float32),
        ],
    )
    def sk(msg_hbm, idx_hbm, zero_hbm, out_hbm, idx_v, msg_v, acc_sh):
        cid = lax.axis_index("c")
        sid = lax.axis_index("s")
        wid = sid * NC + cid
        row0 = sid * RP
        pltpu.sync_copy(zero_hbm.at[pl.ds(row0, RP)], acc_sh.at[pl.ds(row0, RP)])
        plsc.subcore_barrier()
        pltpu.sync_copy(msg_hbm.at[pl.ds(wid * PW, PW)], msg_v)
        pltpu.sync_copy(idx_hbm.at[wid], idx_v)

        @pl.loop(0, C)
        def _(j):
            pltpu.sync_copy(msg_v.at[pl.ds(j * K, K)],
                            acc_sh.at[idx_v.at[j]], add=True)

        plsc.subcore_barrier()
        pltpu.sync_copy(acc_sh.at[pl.ds(row0, RP)], out_hbm.at[cid, pl.ds(row0, RP)])

    return sk(msg, idx_w, zinit)


def _msg(ea, xj, Wa, ba, Wb, bb, Rm, Sm, extra, block_e=4000):
    G = E // block_e
    FP = xj.shape[1]
    Ha = Wa.shape[1]

    def body(ea_ref, xj_ref, wa_ref, ba_ref, wb_ref, bb_ref, r_ref, s_ref,
             ex_ref, out_ref):
        f32 = jnp.float32
        h = jnp.maximum(
            jnp.dot(ea_ref[...], wa_ref[...], preferred_element_type=f32)
            + ba_ref[...], 0.0)
        we = jnp.dot(h, wb_ref[...], preferred_element_type=f32) + bb_ref[...]
        xt = jnp.dot(xj_ref[...], r_ref[...], preferred_element_type=f32)
        out_ref[...] = (
            jnp.dot(we * xt, s_ref[...], preferred_element_type=f32)
            + ex_ref[...])

    return pl.pallas_call(
        body,
        grid=(G,),
        in_specs=[
            pl.BlockSpec((block_e, 16), lambda i: (i, 0)),
            pl.BlockSpec((block_e, FP), lambda i: (i, 0)),
            pl.BlockSpec((16, Ha), lambda i: (0, 0)),
            pl.BlockSpec((1, Ha), lambda i: (0, 0)),
            pl.BlockSpec((Ha, Ha), lambda i: (0, 0)),
            pl.BlockSpec((1, Ha), lambda i: (0, 0)),
            pl.BlockSpec((FP, Ha), lambda i: (0, 0)),
            pl.BlockSpec((Ha, DOUT), lambda i: (0, 0)),
            pl.BlockSpec((1, DOUT), lambda i: (0, 0)),
        ],
        out_specs=pl.BlockSpec((block_e, DOUT), lambda i: (i, 0)),
        out_shape=jax.ShapeDtypeStruct((E, DOUT), jnp.float32),
    )(ea, xj, Wa, ba, Wb, bb, Rm, Sm, extra)


def _post(parts, inv_in, x_cur, root, bias, g, be, c_in, c_out, with_cnt):
    outs = [jax.ShapeDtypeStruct((N, DOUT), jnp.float32)]
    if with_cnt:
        outs.append(jax.ShapeDtypeStruct((N, 1), jnp.float32))

    def body(*refs):
        if with_cnt:
            parts_ref, x_ref, root_ref, bias_ref, g_ref, be_ref, out_ref, inv_ref = refs
        else:
            parts_ref, invin_ref, x_ref, root_ref, bias_ref, g_ref, be_ref, out_ref = refs
        acc = parts_ref[0:N, :] + parts_ref[NPAD:NPAD + N, :]
        if with_cnt:
            inv = 1.0 / jnp.maximum(acc[:, c_out:c_out + 1], 1.0)
            inv_ref[...] = inv
        else:
            inv = invin_ref[...]
        h = (acc[:, 0:c_out] * inv
             + jnp.dot(x_ref[...][:, 0:c_in], root_ref[...],
                       preferred_element_type=jnp.float32)
             + bias_ref[...])
        mu = jnp.mean(h, axis=0, keepdims=True)
        var = jnp.mean((h - mu) ** 2, axis=0, keepdims=True)
        y = g_ref[...] * (h - mu) * lax.rsqrt(var + EPS) + be_ref[...]
        y = jnp.maximum(y, 0.0)
        if c_out < DOUT:
            y = jnp.concatenate(
                [y, jnp.zeros((N, DOUT - c_out), jnp.float32)], axis=1)
        out_ref[...] = y

    ins = [parts] + ([] if with_cnt else [inv_in]) + [x_cur, root, bias, g, be]
    res = pl.pallas_call(body, out_shape=outs)(*ins)
    return res if with_cnt else res[0]


def _mk_RS(c_in, c_out):
    ha = c_in * c_out
    fp = 32 if c_in == 32 else DOUT
    rm = np.zeros((fp, ha), np.float32)
    sm = np.zeros((ha, DOUT), np.float32)
    for i in range(c_in):
        for o in range(c_out):
            rm[i, i * c_out + o] = 1.0
            sm[i * c_out + o, o] = 1.0
    return jnp.asarray(rm), jnp.asarray(sm)


def kernel(x, edge_index, edge_attr, W1a, b1a, W1b, b1b, root1, bias1, g1, be1,
           W2a, b2a, W2b, b2b, root2, bias2, g2, be2,
           W3a, b3a, W3b, b3b, root3, bias3, g3, be3):
    src = edge_index[0].astype(jnp.int32).reshape(NW, C, K)
    dst = edge_index[1].astype(jnp.int32).reshape(NW, C, K)
    zinit = jnp.zeros((NPAD, DOUT), jnp.float32)

    r1, s1 = _mk_RS(32, 8)
    r2, s2 = _mk_RS(8, 4)
    r3, s3 = _mk_RS(4, 16)
    ex1 = np.zeros((1, DOUT), np.float32)
    ex1[0, 8] = 1.0
    ex1 = jnp.asarray(ex1)
    ex0 = jnp.zeros((1, DOUT), jnp.float32)

    def row(v):
        return v.reshape(1, -1)

    xj = _gather(x, src, 32)
    msg = _msg(edge_attr, xj, W1a, row(b1a), W1b, row(b1b), r1, s1, ex1)
    parts = _scatter(msg, dst, zinit)
    h1, invc = _post(parts.reshape(2 * NPAD, DOUT), None, x, root1, row(bias1),
                     row(g1), row(be1), 32, 8, True)

    xj = _gather(h1, src, DOUT)
    msg = _msg(edge_attr, xj, W2a, row(b2a), W2b, row(b2b), r2, s2, ex0)
    parts = _scatter(msg, dst, zinit)
    h2 = _post(parts.reshape(2 * NPAD, DOUT), invc, h1, root2, row(bias2),
               row(g2), row(be2), 8, 4, False)

    xj = _gather(h2, src, DOUT)
    msg = _msg(edge_attr, xj, W3a, row(b3a), W3b, row(b3b), r3, s3, ex0)
    parts = _scatter(msg, dst, zinit)
    h3 = _post(parts.reshape(2 * NPAD, DOUT), invc, h2, root3, row(bias3),
               row(g3), row(be3), 4, 16, False)
    return h3


# speedup vs baseline: 4.0960x; 1.0696x over previous
import functools

import numpy as np
import jax
import jax.numpy as jnp
from jax import lax
from jax.experimental import pallas as pl
from jax.experimental.pallas import tpu as pltpu
from jax.experimental.pallas import tpu_sc as plsc

N = 10000
E = 160000
NC, NS = 2, 16
NW = NC * NS
K = 125
C = E // (NW * K)
PW = C * K
CPB = 8
NB = C // CPB
NPAD = 10240
RP = NPAD // NS
DOUT = 16
EP = E // 8
EPS = 1e-5
_MESH = dict(core_axis_name="c", subcore_axis_name="s")


def _gather16(tables, idx_w):
    nt = len(tables)
    mesh = plsc.VectorSubcoreMesh(**_MESH)

    @functools.partial(
        pl.kernel,
        out_type=[jax.ShapeDtypeStruct((E, DOUT), jnp.float32)] * nt,
        mesh=mesh,
        compiler_params=pltpu.CompilerParams(use_tc_tiling_on_sc=False),
        scratch_types=[pltpu.VMEM((C, K), jnp.int32)]
        + [pltpu.VMEM((CPB * K, DOUT), jnp.float32)] * nt
        + [pltpu.SemaphoreType.DMA],
    )
    def gk(*refs):
        tabs = refs[:nt]
        idx_hbm = refs[nt]
        outs = refs[nt + 1:2 * nt + 1]
        idx_v = refs[2 * nt + 1]
        bufs = refs[2 * nt + 2:3 * nt + 2]
        sem = refs[3 * nt + 2]
        wid = lax.axis_index("s") * NC + lax.axis_index("c")
        base = wid * PW
        pltpu.sync_copy(idx_hbm.at[wid], idx_v)

        @pl.loop(0, NB)
        def _(cc):
            cps = []
            for t in range(CPB):
                for tab, buf in zip(tabs, bufs):
                    cps.append(pltpu.async_copy(
                        tab.at[idx_v.at[cc * CPB + t]],
                        buf.at[pl.ds(t * K, K)], sem))
            for cp in cps:
                cp.wait()
            for buf, out in zip(bufs, outs):
                pltpu.sync_copy(
                    buf, out.at[pl.ds(base + cc * (CPB * K), CPB * K)])

    res = gk(*tables, idx_w)
    if not isinstance(res, (list, tuple)):
        res = [res]
    return [r.reshape(EP, 128) for r in res]


def _scatter(msg_p, idx_w, zinit):
    mesh = plsc.VectorSubcoreMesh(**_MESH)

    @functools.partial(
        pl.kernel,
        out_type=jax.ShapeDtypeStruct((NC, NPAD, DOUT), jnp.float32),
        mesh=mesh,
        compiler_params=pltpu.CompilerParams(use_tc_tiling_on_sc=False),
        scratch_types=[
            pltpu.VMEM((C, K), jnp.int32),
            pltpu.VMEM((PW, DOUT), jnp.float32),
            pltpu.VMEM_SHARED((NPAD, DOUT), jnp.float32),
            pltpu.SemaphoreType.DMA,
        ],
    )
    def sk(msg_hbm, idx_hbm, zero_hbm, out_hbm, idx_v, msg_v, acc_sh, sem):
        cid = lax.axis_index("c")
        sid = lax.axis_index("s")
        wid = sid * NC + cid
        row0 = sid * RP
        pltpu.sync_copy(zero_hbm.at[pl.ds(row0, RP)], acc_sh.at[pl.ds(row0, RP)])
        plsc.subcore_barrier()
        pltpu.sync_copy(msg_hbm.at[pl.ds(wid * PW, PW)], msg_v)
        pltpu.sync_copy(idx_hbm.at[wid], idx_v)

        @pl.loop(0, NB)
        def _(cc):
            cps = []
            for t in range(CPB):
                j = cc * CPB + t
                cps.append(pltpu.async_copy(
                    msg_v.at[pl.ds(j * K, K)], acc_sh.at[idx_v.at[j]], sem,
                    add=True))
            for cp in cps:
                cp.wait()

        plsc.subcore_barrier()
        pltpu.sync_copy(acc_sh.at[pl.ds(row0, RP)], out_hbm.at[cid, pl.ds(row0, RP)])

    return sk(msg_p.reshape(E, DOUT), idx_w, zinit)


def _msg(ea_p, xps, Wa, ba, Wb, bb, Rs, Sm, extra, block_e=8000):
    G = E // block_e
    PR = block_e // 8
    Ha = Wa.shape[1]
    nx = len(xps)

    eqs_np = np.zeros((8 * 128, DOUT), np.float32)
    for q in range(8):
        for c in range(DOUT):
            eqs_np[q * 128 + q * DOUT + c, c] = 1.0
    eqt_np = np.concatenate(
        [eqs_np[q * 128:(q + 1) * 128].T for q in range(8)], axis=1)
    eqs = jnp.asarray(eqs_np)
    eqt = jnp.asarray(eqt_np)

    def body(*refs):
        ea_ref = refs[0]
        xp_refs = refs[1:1 + nx]
        (wa_ref, ba_ref, wb_ref, bb_ref) = refs[1 + nx:5 + nx]
        r_refs = refs[5 + nx:5 + nx + nx]
        (s_ref, ex_ref, eqs_ref, eqt_ref, out_ref) = refs[5 + 2 * nx:]
        f32 = jnp.float32
        dot = functools.partial(jnp.dot, preferred_element_type=f32)
        eap = ea_ref[...]
        xpv = [r[...] for r in xp_refs]
        acc = jnp.zeros((PR, 128), f32)
        for q in range(8):
            eq = eqs_ref[pl.ds(q * 128, 128), :]
            eqt_q = eqt_ref[:, pl.ds(q * 128, 128)]
            ea_q = dot(eap, eq)
            h = jnp.maximum(dot(ea_q, wa_ref[...]) + ba_ref[...], 0.0)
            we = dot(h, wb_ref[...]) + bb_ref[...]
            xt = dot(dot(xpv[0], eq), r_refs[0][...])
            for t in range(1, nx):
                xt = xt + dot(dot(xpv[t], eq), r_refs[t][...])
            y = dot(we * xt, s_ref[...]) + ex_ref[...]
            acc = acc + dot(y, eqt_q)
        out_ref[...] = acc

    full = lambda shape: pl.BlockSpec(shape, lambda i: (0, 0))
    return pl.pallas_call(
        body,
        grid=(G,),
        in_specs=[pl.BlockSpec((PR, 128), lambda i: (i, 0))] * (1 + nx)
        + [full((16, Ha)), full((1, Ha)), full((Ha, Ha)), full((1, Ha))]
        + [full((16, Ha))] * nx
        + [full((Ha, DOUT)), full((1, DOUT)),
           full((8 * 128, DOUT)), full((16, 8 * 128))],
        out_specs=pl.BlockSpec((PR, 128), lambda i: (i, 0)),
        out_shape=jax.ShapeDtypeStruct((EP, 128), jnp.float32),
    )(ea_p, *xps, Wa, ba, Wb, bb, *Rs, Sm, extra, eqs, eqt)


def _post(parts, inv_in, x_cur, root, bias, g, be, c_in, c_out, with_cnt):
    outs = [jax.ShapeDtypeStruct((N, DOUT), jnp.float32)]
    if with_cnt:
        outs.append(jax.ShapeDtypeStruct((N, 1), jnp.float32))

    def body(*refs):
        if with_cnt:
            parts_ref, x_ref, root_ref, bias_ref, g_ref, be_ref, out_ref, inv_ref = refs
        else:
            parts_ref, invin_ref, x_ref, root_ref, bias_ref, g_ref, be_ref, out_ref = refs
        acc = parts_ref[0:N, :] + parts_ref[NPAD:NPAD + N, :]
        if with_cnt:
            inv = 1.0 / jnp.maximum(acc[:, c_out:c_out + 1], 1.0)
            inv_ref[...] = inv
        else:
            inv = invin_ref[...]
        h = (acc[:, 0:c_out] * inv
             + jnp.dot(x_ref[...][:, 0:c_in], root_ref[...],
                       preferred_element_type=jnp.float32)
             + bias_ref[...])
        mu = jnp.mean(h, axis=0, keepdims=True)
        var = jnp.mean((h - mu) ** 2, axis=0, keepdims=True)
        y = g_ref[...] * (h - mu) * lax.rsqrt(var + EPS) + be_ref[...]
        y = jnp.maximum(y, 0.0)
        if c_out < DOUT:
            y = jnp.concatenate(
                [y, jnp.zeros((N, DOUT - c_out), jnp.float32)], axis=1)
        out_ref[...] = y

    ins = [parts] + ([] if with_cnt else [inv_in]) + [x_cur, root, bias, g, be]
    res = pl.pallas_call(body, out_shape=outs)(*ins)
    return res if with_cnt else res[0]


def _mk_RS(c_in, c_out):
    ha = c_in * c_out
    fp = 32 if c_in == 32 else DOUT
    rm = np.zeros((fp, ha), np.float32)
    sm = np.zeros((ha, DOUT), np.float32)
    for i in range(c_in):
        for o in range(c_out):
            rm[i, i * c_out + o] = 1.0
            sm[i * c_out + o, o] = 1.0
    return jnp.asarray(rm), jnp.asarray(sm)


def kernel(x, edge_index, edge_attr, W1a, b1a, W1b, b1b, root1, bias1, g1, be1,
           W2a, b2a, W2b, b2b, root2, bias2, g2, be2,
           W3a, b3a, W3b, b3b, root3, bias3, g3, be3):
    src = edge_index[0].astype(jnp.int32).reshape(NW, C, K)
    dst = edge_index[1].astype(jnp.int32).reshape(NW, C, K)
    zinit = jnp.zeros((NPAD, DOUT), jnp.float32)
    ea_p = edge_attr.reshape(EP, 128)

    r1, s1 = _mk_RS(32, 8)
    r2, s2 = _mk_RS(8, 4)
    r3, s3 = _mk_RS(4, 16)
    ex1 = np.zeros((1, DOUT), np.float32)
    ex1[0, 8] = 1.0
    ex1 = jnp.asarray(ex1)
    ex0 = jnp.zeros((1, DOUT), jnp.float32)

    def row(v):
        return v.reshape(1, -1)

    xa, xb = _gather16([x[:, :16], x[:, 16:]], src)
    msg = _msg(ea_p, [xa, xb], W1a, row(b1a), W1b, row(b1b),
               [r1[:16], r1[16:]], s1, ex1)
    parts = _scatter(msg, dst, zinit)
    h1, invc = _post(parts.reshape(2 * NPAD, DOUT), None, x, root1, row(bias1),
                     row(g1), row(be1), 32, 8, True)

    xj, = _gather16([h1], src)
    msg = _msg(ea_p, [xj], W2a, row(b2a), W2b, row(b2b), [r2], s2, ex0)
    parts = _scatter(msg, dst, zinit)
    h2 = _post(parts.reshape(2 * NPAD, DOUT), invc, h1, root2, row(bias2),
               row(g2), row(be2), 8, 4, False)

    xj, = _gather16([h2], src)
    msg = _msg(ea_p, [xj], W3a, row(b3a), W3b, row(b3b), [r3], s3, ex0)
    parts = _scatter(msg, dst, zinit)
    h3 = _post(parts.reshape(2 * NPAD, DOUT), invc, h2, root3, row(bias3),
               row(g3), row(be3), 4, 16, False)
    return h3


# speedup vs baseline: 6.7461x; 1.6470x over previous
import functools

import numpy as np
import jax
import jax.numpy as jnp
from jax import lax
from jax.experimental import pallas as pl
from jax.experimental.pallas import tpu as pltpu
from jax.experimental.pallas import tpu_sc as plsc

N = 10000
E = 160000
NC, NS = 2, 16
NW = NC * NS
K = 125
C = E // (NW * K)
PW = C * K
CPB = 8
NB = C // CPB
NPAD = 10240
RP = NPAD // NS
DOUT = 16
EP = E // 8
EPS = 1e-5
_MESH = dict(core_axis_name="c", subcore_axis_name="s")


def _gather16(tables, idx_w):
    nt = len(tables)
    mesh = plsc.VectorSubcoreMesh(**_MESH)

    @functools.partial(
        pl.kernel,
        out_type=[jax.ShapeDtypeStruct((E, DOUT), jnp.float32)] * nt,
        mesh=mesh,
        compiler_params=pltpu.CompilerParams(use_tc_tiling_on_sc=False),
        scratch_types=[pltpu.VMEM((C, K), jnp.int32)]
        + [pltpu.VMEM((CPB * K, DOUT), jnp.float32)] * nt
        + [pltpu.SemaphoreType.DMA],
    )
    def gk(*refs):
        tabs = refs[:nt]
        idx_hbm = refs[nt]
        outs = refs[nt + 1:2 * nt + 1]
        idx_v = refs[2 * nt + 1]
        bufs = refs[2 * nt + 2:3 * nt + 2]
        sem = refs[3 * nt + 2]
        wid = lax.axis_index("s") * NC + lax.axis_index("c")
        base = wid * PW
        pltpu.sync_copy(idx_hbm.at[wid], idx_v)

        @pl.loop(0, NB)
        def _(cc):
            cps = []
            for t in range(CPB):
                for tab, buf in zip(tabs, bufs):
                    cps.append(pltpu.async_copy(
                        tab.at[idx_v.at[cc * CPB + t]],
                        buf.at[pl.ds(t * K, K)], sem))
            for cp in cps:
                cp.wait()
            for buf, out in zip(bufs, outs):
                pltpu.sync_copy(
                    buf, out.at[pl.ds(base + cc * (CPB * K), CPB * K)])

    res = gk(*tables, idx_w)
    if not isinstance(res, (list, tuple)):
        res = [res]
    return [r.reshape(EP, 128) for r in res]


def _scatter(msg_p, idx_w, zinit):
    mesh = plsc.VectorSubcoreMesh(**_MESH)

    @functools.partial(
        pl.kernel,
        out_type=jax.ShapeDtypeStruct((NC, NPAD, DOUT), jnp.float32),
        mesh=mesh,
        compiler_params=pltpu.CompilerParams(use_tc_tiling_on_sc=False),
        scratch_types=[
            pltpu.VMEM((C, K), jnp.int32),
            pltpu.VMEM((PW, DOUT), jnp.float32),
            pltpu.VMEM_SHARED((NPAD, DOUT), jnp.float32),
            pltpu.SemaphoreType.DMA,
        ],
    )
    def sk(msg_hbm, idx_hbm, zero_hbm, out_hbm, idx_v, msg_v, acc_sh, sem):
        cid = lax.axis_index("c")
        sid = lax.axis_index("s")
        wid = sid * NC + cid
        row0 = sid * RP
        pltpu.sync_copy(zero_hbm.at[pl.ds(row0, RP)], acc_sh.at[pl.ds(row0, RP)])
        plsc.subcore_barrier()
        pltpu.sync_copy(msg_hbm.at[pl.ds(wid * PW, PW)], msg_v)
        pltpu.sync_copy(idx_hbm.at[wid], idx_v)

        @pl.loop(0, NB)
        def _(cc):
            cps = []
            for t in range(CPB):
                j = cc * CPB + t
                cps.append(pltpu.async_copy(
                    msg_v.at[pl.ds(j * K, K)], acc_sh.at[idx_v.at[j]], sem,
                    add=True))
            for cp in cps:
                cp.wait()

        plsc.subcore_barrier()
        pltpu.sync_copy(acc_sh.at[pl.ds(row0, RP)], out_hbm.at[cid, pl.ds(row0, RP)])

    return sk(msg_p.reshape(E, DOUT), idx_w, zinit)


def _eq_consts():
    eqs_np = np.zeros((8 * 128, DOUT), np.float32)
    for q in range(8):
        for c in range(DOUT):
            eqs_np[q * 128 + q * DOUT + c, c] = 1.0
    eqt_np = np.concatenate(
        [eqs_np[q * 128:(q + 1) * 128].T for q in range(8)], axis=1)
    return jnp.asarray(eqs_np), jnp.asarray(eqt_np)


def _msg(ea_p, xps, Wa, ba, Wb, bb, Rs, Sm, extra, block_e=16000):
    G = E // block_e
    PR = block_e // 8
    Ha = Wa.shape[1]
    nx = len(xps)
    eqs, eqt = _eq_consts()
    f32 = jnp.float32
    waq = jnp.dot(eqs, Wa, preferred_element_type=f32)
    rq = [jnp.dot(eqs, r, preferred_element_type=f32) for r in Rs]
    sq = jnp.dot(Sm, eqt, preferred_element_type=f32)
    exp = jnp.tile(extra, (1, 8))

    def body(*refs):
        ea_ref = refs[0]
        xp_refs = refs[1:1 + nx]
        (waq_ref, ba_ref, wb_ref, bb_ref) = refs[1 + nx:5 + nx]
        rq_refs = refs[5 + nx:5 + 2 * nx]
        (sq_ref, ex_ref, out_ref) = refs[5 + 2 * nx:]
        dot = functools.partial(jnp.dot, preferred_element_type=f32)
        eap = ea_ref[...]
        xpv = [r[...] for r in xp_refs]
        acc = ex_ref[...] + jnp.zeros((PR, 128), f32)
        for q in range(8):
            h = jnp.maximum(
                dot(eap, waq_ref[pl.ds(q * 128, 128), :]) + ba_ref[...], 0.0)
            we = dot(h, wb_ref[...]) + bb_ref[...]
            xt = dot(xpv[0], rq_refs[0][pl.ds(q * 128, 128), :])
            for t in range(1, nx):
                xt = xt + dot(xpv[t], rq_refs[t][pl.ds(q * 128, 128), :])
            acc = acc + dot(we * xt, sq_ref[:, pl.ds(q * 128, 128)])
        out_ref[...] = acc

    full = lambda shape: pl.BlockSpec(shape, lambda i: (0, 0))
    return pl.pallas_call(
        body,
        grid=(G,),
        in_specs=[pl.BlockSpec((PR, 128), lambda i: (i, 0))] * (1 + nx)
        + [full((8 * 128, Ha)), full((1, Ha)), full((Ha, Ha)), full((1, Ha))]
        + [full((8 * 128, Ha))] * nx
        + [full((Ha, 8 * 128)), full((1, 128))],
        out_specs=pl.BlockSpec((PR, 128), lambda i: (i, 0)),
        out_shape=jax.ShapeDtypeStruct((EP, 128), jnp.float32),
    )(ea_p, *xps, waq, ba, Wb, bb, *rq, sq, exp)


def _post(parts, inv_in, x_cur, root, bias, g, be, c_in, c_out, with_cnt):
    outs = [jax.ShapeDtypeStruct((N, DOUT), jnp.float32)]
    if with_cnt:
        outs.append(jax.ShapeDtypeStruct((N, 1), jnp.float32))

    def body(*refs):
        if with_cnt:
            parts_ref, x_ref, root_ref, bias_ref, g_ref, be_ref, out_ref, inv_ref = refs
        else:
            parts_ref, invin_ref, x_ref, root_ref, bias_ref, g_ref, be_ref, out_ref = refs
        acc = parts_ref[0:N, :] + parts_ref[NPAD:NPAD + N, :]
        if with_cnt:
            inv = 1.0 / jnp.maximum(acc[:, c_out:c_out + 1], 1.0)
            inv_ref[...] = inv
        else:
            inv = invin_ref[...]
        h = (acc[:, 0:c_out] * inv
             + jnp.dot(x_ref[...][:, 0:c_in], root_ref[...],
                       preferred_element_type=jnp.float32)
             + bias_ref[...])
        mu = jnp.mean(h, axis=0, keepdims=True)
        var = jnp.mean((h - mu) ** 2, axis=0, keepdims=True)
        y = g_ref[...] * (h - mu) * lax.rsqrt(var + EPS) + be_ref[...]
        y = jnp.maximum(y, 0.0)
        if c_out < DOUT:
            y = jnp.concatenate(
                [y, jnp.zeros((N, DOUT - c_out), jnp.float32)], axis=1)
        out_ref[...] = y

    ins = [parts] + ([] if with_cnt else [inv_in]) + [x_cur, root, bias, g, be]
    res = pl.pallas_call(body, out_shape=outs)(*ins)
    return res if with_cnt else res[0]


def _mk_RS(c_in, c_out):
    ha = c_in * c_out
    fp = 32 if c_in == 32 else DOUT
    rm = np.zeros((fp, ha), np.float32)
    sm = np.zeros((ha, DOUT), np.float32)
    for i in range(c_in):
        for o in range(c_out):
            rm[i, i * c_out + o] = 1.0
            sm[i * c_out + o, o] = 1.0
    return jnp.asarray(rm), jnp.asarray(sm)


def kernel(x, edge_index, edge_attr, W1a, b1a, W1b, b1b, root1, bias1, g1, be1,
           W2a, b2a, W2b, b2b, root2, bias2, g2, be2,
           W3a, b3a, W3b, b3b, root3, bias3, g3, be3):
    src = edge_index[0].astype(jnp.int32).reshape(NW, C, K)
    dst = edge_index[1].astype(jnp.int32).reshape(NW, C, K)
    zinit = jnp.zeros((NPAD, DOUT), jnp.float32)
    ea_p = edge_attr.reshape(EP, 128)

    r1, s1 = _mk_RS(32, 8)
    r2, s2 = _mk_RS(8, 4)
    r3, s3 = _mk_RS(4, 16)
    ex1 = np.zeros((1, DOUT), np.float32)
    ex1[0, 8] = 1.0
    ex1 = jnp.asarray(ex1)
    ex0 = jnp.zeros((1, DOUT), jnp.float32)

    def row(v):
        return v.reshape(1, -1)

    xa, xb = _gather16([x[:, :16], x[:, 16:]], src)
    msg = _msg(ea_p, [xa, xb], W1a, row(b1a), W1b, row(b1b),
               [r1[:16], r1[16:]], s1, ex1)
    parts = _scatter(msg, dst, zinit)
    h1, invc = _post(parts.reshape(2 * NPAD, DOUT), None, x, root1, row(bias1),
                     row(g1), row(be1), 32, 8, True)

    xj, = _gather16([h1], src)
    msg = _msg(ea_p, [xj], W2a, row(b2a), W2b, row(b2b), [r2], s2, ex0)
    parts = _scatter(msg, dst, zinit)
    h2 = _post(parts.reshape(2 * NPAD, DOUT), invc, h1, root2, row(bias2),
               row(g2), row(be2), 8, 4, False)

    xj, = _gather16([h2], src)
    msg = _msg(ea_p, [xj], W3a, row(b3a), W3b, row(b3b), [r3], s3, ex0)
    parts = _scatter(msg, dst, zinit)
    h3 = _post(parts.reshape(2 * NPAD, DOUT), invc, h2, root3, row(bias3),
               row(g3), row(be3), 4, 16, False)
    return h3
